# baseline (speedup 1.0000x reference)
import functools

import jax
import jax.numpy as jnp
from jax.experimental import pallas as pl
from jax.experimental.pallas import tpu as pltpu
from jax.experimental.pallas import tpu_sc as plsc

OBS_DIM = 64
N_CODE_EACH = 512
CODE_DIM = 64
BATCH = 1024
HIDDEN = 256
N_CODE_TOTAL = OBS_DIM * N_CODE_EACH
REP_DIM = OBS_DIM * CODE_DIM

J_CHUNKS = 2
JC = OBS_DIM // J_CHUNKS
KH = N_CODE_EACH // 2


def _argmin_body(j0, emb_ref, obsT_ref, encwT_ref, encbT_ref,
                 fidx2_ref, par_ref, pair_ref):
    j = pl.program_id(0)
    jg = j + j0
    Wt = emb_ref[...].T
    pair_ref[...] = jnp.concatenate([Wt[:KH, :], Wt[KH:, :]], axis=1)
    ohj = jax.lax.broadcasted_iota(jnp.int32, (CODE_DIM, OBS_DIM), 1) == jg
    ewc = jnp.sum(jnp.where(ohj, encwT_ref[...], 0.0), axis=1, keepdims=True)
    ebc = jnp.sum(jnp.where(ohj, encbT_ref[...], 0.0), axis=1, keepdims=True)
    ob = obsT_ref[pl.ds(jg, 1), :]
    zT = ob * ewc + ebc
    cross = jnp.dot(Wt, zT, preferred_element_type=jnp.float32)
    z2 = jnp.sum(zT * zT, axis=0, keepdims=True)
    w2 = jnp.sum(Wt * Wt, axis=1, keepdims=True)
    dists = (z2 + w2) - 2.0 * cross
    m = jnp.min(dists, axis=0, keepdims=True)
    kio = jax.lax.broadcasted_iota(jnp.int32, dists.shape, 0)
    cand = jnp.where(dists == m, kio, N_CODE_EACH)
    idx = jnp.min(cand, axis=0, keepdims=True)
    fidx2_ref[pl.ds(j, 1), :] = (idx & (KH - 1)) + j * KH
    par_ref[pl.ds(j, 1), :] = idx >> 8


def _ze_body(obsT_ref, encwT_ref, encbT_ref, ze_ref):
    ze3 = (obsT_ref[...][None, :, :] * encwT_ref[...][:, :, None]
           + encbT_ref[...][:, :, None])
    ze_ref[...] = ze3.reshape(8 * OBS_DIM, BATCH)


def _sel_half(q2_ref, par_ref):
    q2 = q2_ref[...]
    par3 = par_ref[...][:, :, None]
    return jnp.where(par3 == 0, q2[:, :, :CODE_DIM], q2[:, :, CODE_DIM:])


def _decoder0_body(q2a_ref, para_ref, w1a_ref, p0_ref, emba_ref):
    sel = _sel_half(q2a_ref, para_ref)
    emba_ref[...] = sel
    p0_ref[...] = jnp.dot(sel.reshape(sel.shape[0], REP_DIM // 2),
                          w1a_ref[...], preferred_element_type=jnp.float32)


def _decoder1_body(q2b_ref, parb_ref, p0_ref, w1b_ref, b1_ref,
                   w2_ref, b2_ref, recon_ref, embb_ref):
    sel = _sel_half(q2b_ref, parb_ref)
    embb_ref[...] = sel
    h = (p0_ref[...]
         + jnp.dot(sel.reshape(sel.shape[0], REP_DIM // 2), w1b_ref[...],
                   preferred_element_type=jnp.float32)
         + b1_ref[...])
    h = jnp.maximum(h, 0.0)
    recon_ref[...] = jnp.dot(h, w2_ref[...],
                             preferred_element_type=jnp.float32) + b2_ref[...]


def _sc_gather(table, fidx2):
    n_idx = fidx2.shape[1]
    window = 256

    @pl.kernel(
        out_type=jax.ShapeDtypeStruct((n_idx, 2 * CODE_DIM), table.dtype),
        mesh=plsc.VectorSubcoreMesh(core_axis_name="core",
                                    subcore_axis_name="subcore"),
    )
    def kern(x_hbm, i_hbm, o_hbm):
        def body(i_vmem, o_vmem):
            pltpu.sync_copy(x_hbm.at[i_vmem.at[0]], o_vmem)

        pltpu.emit_pipeline(
            body,
            grid=(n_idx // window,),
            in_specs=[pl.BlockSpec((1, window), index_map=lambda i: (0, i))],
            out_specs=[pl.BlockSpec((window, 2 * CODE_DIM),
                                    index_map=lambda i: (i, 0))],
            core_axis_name=("core", "subcore"),
            dimension_semantics=(pltpu.PARALLEL,),
        )(i_hbm, o_hbm)

    return kern(table, fidx2)


def _argmin_chunk(c, emb_weight, obsT, encwT, encbT):
    return pl.pallas_call(
        functools.partial(_argmin_body, c * JC),
        grid=(JC,),
        in_specs=[
            pl.BlockSpec((CODE_DIM, N_CODE_EACH),
                         lambda j, c=c: (0, c * JC + j)),
            pl.BlockSpec((OBS_DIM, BATCH), lambda j: (0, 0)),
            pl.BlockSpec((CODE_DIM, OBS_DIM), lambda j: (0, 0)),
            pl.BlockSpec((CODE_DIM, OBS_DIM), lambda j: (0, 0)),
        ],
        out_specs=[
            pl.BlockSpec((JC, BATCH), lambda j: (0, 0)),
            pl.BlockSpec((JC, BATCH), lambda j: (0, 0)),
            pl.BlockSpec((KH, 2 * CODE_DIM), lambda j: (j, 0)),
        ],
        out_shape=[
            jax.ShapeDtypeStruct((JC, BATCH), jnp.int32),
            jax.ShapeDtypeStruct((JC, BATCH), jnp.int32),
            jax.ShapeDtypeStruct((JC * KH, 2 * CODE_DIM), jnp.float32),
        ],
    )(emb_weight, obsT, encwT, encbT)


def kernel(obs, enc_w, enc_b, emb_weight, dec_w1, dec_b1, dec_w2, dec_b2):
    obsT = obs.T
    encwT = enc_w.T
    encbT = enc_b.T

    q2vs, pars = [], []
    for c in range(J_CHUNKS):
        fidx2T_c, parT_c, pair_c = _argmin_chunk(
            c, emb_weight, obsT, encwT, encbT)
        fidx2_c = fidx2T_c.T.reshape(1, BATCH * JC)
        q2_c = _sc_gather(pair_c, fidx2_c)
        q2vs.append(q2_c.reshape(BATCH, JC, 2 * CODE_DIM))
        pars.append(parT_c.T)

    ze2 = pl.pallas_call(
        _ze_body,
        grid=(CODE_DIM // 8,),
        in_specs=[
            pl.BlockSpec((OBS_DIM, BATCH), lambda i: (0, 0)),
            pl.BlockSpec((8, OBS_DIM), lambda i: (i, 0)),
            pl.BlockSpec((8, OBS_DIM), lambda i: (i, 0)),
        ],
        out_specs=pl.BlockSpec((8 * OBS_DIM, BATCH), lambda i: (i, 0)),
        out_shape=jax.ShapeDtypeStruct((REP_DIM, BATCH), jnp.float32),
    )(obsT, encwT, encbT)

    bb = 128
    p0, embA = pl.pallas_call(
        _decoder0_body,
        grid=(BATCH // bb,),
        in_specs=[
            pl.BlockSpec((bb, JC, 2 * CODE_DIM), lambda i: (i, 0, 0)),
            pl.BlockSpec((bb, JC), lambda i: (i, 0)),
            pl.BlockSpec((REP_DIM // 2, HIDDEN), lambda i: (0, 0)),
        ],
        out_specs=[
            pl.BlockSpec((bb, HIDDEN), lambda i: (i, 0)),
            pl.BlockSpec((bb, JC, CODE_DIM), lambda i: (i, 0, 0)),
        ],
        out_shape=[
            jax.ShapeDtypeStruct((BATCH, HIDDEN), jnp.float32),
            jax.ShapeDtypeStruct((BATCH, JC, CODE_DIM), jnp.float32),
        ],
    )(q2vs[0], pars[0], dec_w1[:REP_DIM // 2])

    recon, embB = pl.pallas_call(
        _decoder1_body,
        grid=(BATCH // bb,),
        in_specs=[
            pl.BlockSpec((bb, JC, 2 * CODE_DIM), lambda i: (i, 0, 0)),
            pl.BlockSpec((bb, JC), lambda i: (i, 0)),
            pl.BlockSpec((bb, HIDDEN), lambda i: (i, 0)),
            pl.BlockSpec((REP_DIM // 2, HIDDEN), lambda i: (0, 0)),
            pl.BlockSpec((1, HIDDEN), lambda i: (0, 0)),
            pl.BlockSpec((HIDDEN, OBS_DIM), lambda i: (0, 0)),
            pl.BlockSpec((1, OBS_DIM), lambda i: (0, 0)),
        ],
        out_specs=[
            pl.BlockSpec((bb, OBS_DIM), lambda i: (i, 0)),
            pl.BlockSpec((bb, JC, CODE_DIM), lambda i: (i, 0, 0)),
        ],
        out_shape=[
            jax.ShapeDtypeStruct((BATCH, OBS_DIM), jnp.float32),
            jax.ShapeDtypeStruct((BATCH, JC, CODE_DIM), jnp.float32),
        ],
    )(q2vs[1], pars[1], p0, dec_w1[REP_DIM // 2:],
      dec_b1.reshape(1, HIDDEN), dec_w2, dec_b2.reshape(1, OBS_DIM))

    ze = jnp.transpose(ze2.reshape(CODE_DIM, OBS_DIM, BATCH), (2, 0, 1))
    emb = jnp.swapaxes(jnp.concatenate([embA, embB], axis=1), 1, 2)
    return recon, ze, emb


# speedup vs baseline: 1.1274x; 1.1274x over previous
import functools

import jax
import jax.numpy as jnp
from jax.experimental import pallas as pl
from jax.experimental.pallas import tpu as pltpu
from jax.experimental.pallas import tpu_sc as plsc

OBS_DIM = 64
N_CODE_EACH = 512
CODE_DIM = 64
BATCH = 1024
HIDDEN = 256
N_CODE_TOTAL = OBS_DIM * N_CODE_EACH
REP_DIM = OBS_DIM * CODE_DIM

J_CHUNKS = 2
JC = OBS_DIM // J_CHUNKS
KH = N_CODE_EACH // 2


def _argmin_body(j0, emb_ref, obsT_ref, encwT_ref, encbT_ref,
                 fidx2_ref, par_ref, pair_ref):
    j = pl.program_id(0)
    jg = j + j0
    Wt = emb_ref[...].T
    pair_ref[...] = jnp.concatenate([Wt[:KH, :], Wt[KH:, :]], axis=1)
    ohj = jax.lax.broadcasted_iota(jnp.int32, (CODE_DIM, OBS_DIM), 1) == jg
    ewc = jnp.sum(jnp.where(ohj, encwT_ref[...], 0.0), axis=1, keepdims=True)
    ebc = jnp.sum(jnp.where(ohj, encbT_ref[...], 0.0), axis=1, keepdims=True)
    ob = obsT_ref[pl.ds(jg, 1), :]
    zT = ob * ewc + ebc
    cross = jnp.dot(Wt, zT, preferred_element_type=jnp.float32)
    z2 = jnp.sum(zT * zT, axis=0, keepdims=True)
    w2 = jnp.sum(Wt * Wt, axis=1, keepdims=True)
    dists = (z2 + w2) - 2.0 * cross
    m = jnp.min(dists, axis=0, keepdims=True)
    kio = jax.lax.broadcasted_iota(jnp.int32, dists.shape, 0)
    cand = jnp.where(dists == m, kio, N_CODE_EACH)
    idx = jnp.min(cand, axis=0, keepdims=True)
    fidx2_ref[pl.ds(j, 1), :] = (idx & (KH - 1)) + j * KH
    par_ref[pl.ds(j, 1), :] = idx >> 8


def _ze_body(obsT_ref, encwT_ref, encbT_ref, ze_ref):
    ze3 = (obsT_ref[...][None, :, :] * encwT_ref[...][:, :, None]
           + encbT_ref[...][:, :, None])
    ze_ref[...] = ze3.reshape(8 * OBS_DIM, BATCH)


def _decoder_body(q2a_ref, q2b_ref, para_ref, parb_ref,
                  w1_ref, b1_ref, w2_ref, b2_ref,
                  recon_ref, emb_ref):
    sels = []
    for q2_ref, par_ref in ((q2a_ref, para_ref), (q2b_ref, parb_ref)):
        q2 = q2_ref[...]
        par3 = par_ref[...][:, :, None]
        sels.append(jnp.where(par3 == 0,
                              q2[:, :, :CODE_DIM], q2[:, :, CODE_DIM:]))
    sel = jnp.concatenate(sels, axis=1)
    emb_ref[...] = sel
    qf = sel.reshape(sel.shape[0], REP_DIM)
    h = jnp.dot(qf, w1_ref[...],
                preferred_element_type=jnp.float32) + b1_ref[...]
    h = jnp.maximum(h, 0.0)
    recon_ref[...] = jnp.dot(h, w2_ref[...],
                             preferred_element_type=jnp.float32) + b2_ref[...]


def _sc_gather(table, fidx2):
    n_idx = fidx2.shape[1]
    window = 256

    @pl.kernel(
        out_type=jax.ShapeDtypeStruct((n_idx, 2 * CODE_DIM), table.dtype),
        mesh=plsc.VectorSubcoreMesh(core_axis_name="core",
                                    subcore_axis_name="subcore"),
    )
    def kern(x_hbm, i_hbm, o_hbm):
        def body(i_vmem, o_vmem):
            pltpu.sync_copy(x_hbm.at[i_vmem.at[0]], o_vmem)

        pltpu.emit_pipeline(
            body,
            grid=(n_idx // window,),
            in_specs=[pl.BlockSpec((1, window), index_map=lambda i: (0, i))],
            out_specs=[pl.BlockSpec((window, 2 * CODE_DIM),
                                    index_map=lambda i: (i, 0))],
            core_axis_name=("core", "subcore"),
            dimension_semantics=(pltpu.PARALLEL,),
        )(i_hbm, o_hbm)

    return kern(table, fidx2)


def _argmin_chunk(c, emb_weight, obsT, encwT, encbT):
    return pl.pallas_call(
        functools.partial(_argmin_body, c * JC),
        grid=(JC,),
        in_specs=[
            pl.BlockSpec((CODE_DIM, N_CODE_EACH),
                         lambda j, c=c: (0, c * JC + j)),
            pl.BlockSpec((OBS_DIM, BATCH), lambda j: (0, 0)),
            pl.BlockSpec((CODE_DIM, OBS_DIM), lambda j: (0, 0)),
            pl.BlockSpec((CODE_DIM, OBS_DIM), lambda j: (0, 0)),
        ],
        out_specs=[
            pl.BlockSpec((JC, BATCH), lambda j: (0, 0)),
            pl.BlockSpec((JC, BATCH), lambda j: (0, 0)),
            pl.BlockSpec((KH, 2 * CODE_DIM), lambda j: (j, 0)),
        ],
        out_shape=[
            jax.ShapeDtypeStruct((JC, BATCH), jnp.int32),
            jax.ShapeDtypeStruct((JC, BATCH), jnp.int32),
            jax.ShapeDtypeStruct((JC * KH, 2 * CODE_DIM), jnp.float32),
        ],
    )(emb_weight, obsT, encwT, encbT)


def kernel(obs, enc_w, enc_b, emb_weight, dec_w1, dec_b1, dec_w2, dec_b2):
    obsT = obs.T
    encwT = enc_w.T
    encbT = enc_b.T

    q2vs, pars = [], []
    for c in range(J_CHUNKS):
        fidx2T_c, parT_c, pair_c = _argmin_chunk(
            c, emb_weight, obsT, encwT, encbT)
        fidx2_c = fidx2T_c.T.reshape(1, BATCH * JC)
        q2_c = _sc_gather(pair_c, fidx2_c)
        q2vs.append(q2_c.reshape(BATCH, JC, 2 * CODE_DIM))
        pars.append(parT_c.T)

    ze2 = pl.pallas_call(
        _ze_body,
        grid=(CODE_DIM // 8,),
        in_specs=[
            pl.BlockSpec((OBS_DIM, BATCH), lambda i: (0, 0)),
            pl.BlockSpec((8, OBS_DIM), lambda i: (i, 0)),
            pl.BlockSpec((8, OBS_DIM), lambda i: (i, 0)),
        ],
        out_specs=pl.BlockSpec((8 * OBS_DIM, BATCH), lambda i: (i, 0)),
        out_shape=jax.ShapeDtypeStruct((REP_DIM, BATCH), jnp.float32),
    )(obsT, encwT, encbT)

    bb = 128
    recon, embJD = pl.pallas_call(
        _decoder_body,
        grid=(BATCH // bb,),
        in_specs=[
            pl.BlockSpec((bb, JC, 2 * CODE_DIM), lambda i: (i, 0, 0)),
            pl.BlockSpec((bb, JC, 2 * CODE_DIM), lambda i: (i, 0, 0)),
            pl.BlockSpec((bb, JC), lambda i: (i, 0)),
            pl.BlockSpec((bb, JC), lambda i: (i, 0)),
            pl.BlockSpec((REP_DIM, HIDDEN), lambda i: (0, 0)),
            pl.BlockSpec((1, HIDDEN), lambda i: (0, 0)),
            pl.BlockSpec((HIDDEN, OBS_DIM), lambda i: (0, 0)),
            pl.BlockSpec((1, OBS_DIM), lambda i: (0, 0)),
        ],
        out_specs=[
            pl.BlockSpec((bb, OBS_DIM), lambda i: (i, 0)),
            pl.BlockSpec((bb, OBS_DIM, CODE_DIM), lambda i: (i, 0, 0)),
        ],
        out_shape=[
            jax.ShapeDtypeStruct((BATCH, OBS_DIM), jnp.float32),
            jax.ShapeDtypeStruct((BATCH, OBS_DIM, CODE_DIM), jnp.float32),
        ],
    )(q2vs[0], q2vs[1], pars[0], pars[1], dec_w1,
      dec_b1.reshape(1, HIDDEN), dec_w2, dec_b2.reshape(1, OBS_DIM))

    ze = jnp.transpose(ze2.reshape(CODE_DIM, OBS_DIM, BATCH), (2, 0, 1))
    emb = jnp.swapaxes(embJD, 1, 2)
    return recon, ze, emb
